# baseline (speedup 1.0000x reference)
import functools

import jax
import jax.numpy as jnp
from jax import lax
from jax.experimental import pallas as pl
from jax.experimental.pallas import tpu as pltpu
from jax.experimental.pallas import tpu_sc as plsc

V = 10000
E = 320000
D = 128

NC = 2
NS = 16
NW = NC * NS
EPW = E // NW
CH = 80
NCH = EPW // CH
SPLIT_NCH = (15, 40, 40, 30)
SPLIT_EPW = tuple(n * CH for n in SPLIT_NCH)
SPLIT_E = tuple(NW * epw for epw in SPLIT_EPW)
SPLIT_E0 = tuple(sum(SPLIT_E[:i]) for i in range(len(SPLIT_E)))
VP = 10240
VPS = VP // NS
ZR = 128


def _proj_body(h_ref, wa_ref, wb_ref, be1_ref, p_ref, q_ref):
    h = h_ref[...]
    p_ref[...] = jnp.dot(h, wa_ref[...], preferred_element_type=jnp.float32)
    q_ref[...] = (jnp.dot(h, wb_ref[...], preferred_element_type=jnp.float32)
                  + be1_ref[...])


def _edge_mlp_body(x1_ref, w2_ref, b2_ref, w3_ref, b3_ref, msg_ref):
    x1 = x1_ref[...].astype(jnp.bfloat16)
    x2 = jnp.maximum(
        jnp.dot(x1, w2_ref[...], preferred_element_type=jnp.float32)
        + b2_ref[...], 0.0).astype(jnp.bfloat16)
    msg_ref[...] = (jnp.dot(x2, w3_ref[...], preferred_element_type=jnp.float32)
                    + b3_ref[...])


def _make_node_mlp_body(nagg):
    def body(h_ref, *refs):
        aggs = refs[:nagg]
        wna_ref, wnb_ref, bn1_ref, wn2_ref, bn2_ref, out_ref = refs[nagg:]
        agg = aggs[0][0]
        for a in aggs[1:]:
            agg = agg + a[0]
        _node_mlp_core(h_ref, agg, wna_ref, wnb_ref, bn1_ref, wn2_ref,
                       bn2_ref, out_ref)
    return body


def _node_mlp_core(h_ref, agg, wna_ref, wnb_ref, bn1_ref,
                   wn2_ref, bn2_ref, out_ref):
    y = jnp.maximum(
        jnp.dot(h_ref[...], wna_ref[...], preferred_element_type=jnp.float32)
        + jnp.dot(agg, wnb_ref[...], preferred_element_type=jnp.float32)
        + bn1_ref[...], 0.0)
    out_ref[...] = (jnp.dot(y, wn2_ref[...], preferred_element_type=jnp.float32)
                    + bn2_ref[...])


def _full(shape):
    return pl.BlockSpec(shape, lambda i: (0,) * len(shape))


def _make_gather_body(e0, epw, nch):
    def body(p_hbm, q_hbm, src_hbm, dst_hbm, x1_hbm,
             sidx, didx, pbuf0, qbuf0, pbuf1, qbuf1, obuf0, obuf1,
             gsem0, gsem1, wsem0, wsem1):
        c = lax.axis_index("c")
        s = lax.axis_index("s")
        wid = s * NC + c
        gbase = e0 + wid * epw
        lbase = wid * epw
        pbufs, qbufs = (pbuf0, pbuf1), (qbuf0, qbuf1)
        obufs = (obuf0, obuf1)
        gsems, wsems = (gsem0, gsem1), (wsem0, wsem1)

        pltpu.sync_copy(src_hbm.at[pl.ds(gbase, epw)], sidx)
        pltpu.sync_copy(dst_hbm.at[pl.ds(gbase, epw)], didx)

        def fire_gather(ch, b):
            pltpu.async_copy(p_hbm.at[sidx.at[pl.ds(ch * CH, CH)]], pbufs[b],
                             gsems[b])
            pltpu.async_copy(q_hbm.at[didx.at[pl.ds(ch * CH, CH)]], qbufs[b],
                             gsems[b])

        def wait_gather(b):
            pltpu.make_async_copy(p_hbm.at[sidx.at[pl.ds(0, CH)]], pbufs[b],
                                  gsems[b]).wait()
            pltpu.make_async_copy(q_hbm.at[didx.at[pl.ds(0, CH)]], qbufs[b],
                                  gsems[b]).wait()

        def fire_wb(ch, b):
            off = lbase + ch * CH
            pltpu.async_copy(obufs[b], x1_hbm.at[pl.ds(off, CH)], wsems[b])

        def wait_wb(b):
            pltpu.make_async_copy(obufs[b], x1_hbm.at[pl.ds(lbase, CH)],
                                  wsems[b]).wait()

        def compute(b):
            pb, qb, ob = pbufs[b], qbufs[b], obufs[b]

            def row(r, carry):
                for j in range(D // 16):
                    sl = pl.ds(j * 16, 16)
                    ob[r, sl] = jnp.maximum(pb[r, sl] + qb[r, sl], 0.0)
                return carry

            lax.fori_loop(0, CH, row, 0)

        fire_gather(0, 0)

        def super_step(i, carry):
            for b in range(2):
                ch = 2 * i + b
                wait_gather(b)
                fire_gather(ch + 1, 1 - b)

                @pl.when(ch >= 2)
                def _():
                    wait_wb(b)

                compute(b)
                fire_wb(ch, b)
            return carry

        lax.fori_loop(0, (nch - 1) // 2, super_step, 0)
        if nch % 2 == 1:
            wait_gather(0)
            wait_wb(0)
            compute(0)
            fire_wb(nch - 1, 0)
            wait_wb(1)
            wait_wb(0)
        else:
            wait_gather(0)
            fire_gather(nch - 1, 1)
            wait_wb(0)
            compute(0)
            fire_wb(nch - 2, 0)
            wait_gather(1)
            wait_wb(1)
            compute(1)
            fire_wb(nch - 1, 1)
            wait_wb(0)
            wait_wb(1)

    return body


def _make_scatter_body(e0, epw, nch):
    def body(msg_hbm, dst_hbm, out_hbm,
             didx0, didx1, mbuf0, mbuf1, zbuf, agg_sh,
             lsem0, lsem1, ssem0, ssem1):
        c = lax.axis_index("c")
        s = lax.axis_index("s")
        wid = s * NC + c
        gbase = e0 + wid * epw
        lbase = wid * epw
        didxs, mbufs = (didx0, didx1), (mbuf0, mbuf1)
        lsems, ssems = (lsem0, lsem1), (ssem0, ssem1)

        def fire_load(ch, b):
            pltpu.async_copy(dst_hbm.at[pl.ds(gbase + ch * CH, CH)],
                             didxs[b], lsems[b])
            pltpu.async_copy(msg_hbm.at[pl.ds(lbase + ch * CH, CH)],
                             mbufs[b], lsems[b])

        def wait_load(b):
            pltpu.make_async_copy(dst_hbm.at[pl.ds(gbase, CH)], didxs[b],
                                  lsems[b]).wait()
            pltpu.make_async_copy(msg_hbm.at[pl.ds(lbase, CH)], mbufs[b],
                                  lsems[b]).wait()

        def fire_scat(b):
            pltpu.async_copy(mbufs[b], agg_sh.at[didxs[b]], ssems[b],
                             add=True)

        def wait_scat(b):
            pltpu.make_async_copy(mbufs[b], agg_sh.at[didxs[b]],
                                  ssems[b]).wait()

        zero = jnp.zeros((16,), jnp.float32)

        def zrow(i, carry):
            for j in range(D // 16):
                zbuf[i, pl.ds(j * 16, 16)] = zero
            return carry

        fire_load(0, 0)
        lax.fori_loop(0, ZR, zrow, 0)
        for k in range(VPS // ZR):
            pltpu.sync_copy(zbuf, agg_sh.at[pl.ds(s * VPS + k * ZR, ZR)])
        plsc.subcore_barrier()

        def super_step(i, carry):
            for b in range(2):
                ch = 2 * i + b
                wait_load(b)
                fire_scat(b)

                @pl.when(ch >= 1)
                def _():
                    wait_scat(1 - b)

                fire_load(ch + 1, 1 - b)
            return carry

        lax.fori_loop(0, (nch - 1) // 2, super_step, 0)
        if nch % 2 == 1:
            wait_load(0)
            fire_scat(0)
            wait_scat(1)
            wait_scat(0)
        else:
            wait_load(0)
            fire_scat(0)
            wait_scat(1)
            fire_load(nch - 1, 1)
            wait_load(1)
            fire_scat(1)
            wait_scat(0)
            wait_scat(1)
        plsc.subcore_barrier()
        pltpu.sync_copy(agg_sh.at[pl.ds(s * VPS, VPS)],
                        out_hbm.at[c, pl.ds(s * VPS, VPS)])

    return body


@jax.jit
def kernel(h, edge_index, We1, be1, We2, be2, We3, be3, Wn1, bn1, Wn2, bn2):
    src = edge_index[0].astype(jnp.int32)
    dst = edge_index[1].astype(jnp.int32)
    be1r = be1.reshape(1, D)
    be2r = be2.reshape(1, D)
    be3r = be3.reshape(1, D)
    bn1r = bn1.reshape(1, D)
    bn2r = bn2.reshape(1, D)

    BV = 1000
    p, q = pl.pallas_call(
        _proj_body,
        grid=(V // BV,),
        in_specs=[
            pl.BlockSpec((BV, D), lambda i: (i, 0)),
            _full((D, D)), _full((D, D)), _full((1, D)),
        ],
        out_specs=[
            pl.BlockSpec((BV, D), lambda i: (i, 0)),
            pl.BlockSpec((BV, D), lambda i: (i, 0)),
        ],
        out_shape=[
            jax.ShapeDtypeStruct((V, D), jnp.float32),
            jax.ShapeDtypeStruct((V, D), jnp.float32),
        ],
    )(h, We1[:D], We1[D:], be1r)

    mesh = plsc.VectorSubcoreMesh(core_axis_name="c", subcore_axis_name="s")

    def gather_call(e0, epw, nch):
        return pl.kernel(
            _make_gather_body(e0, epw, nch),
            out_type=jax.ShapeDtypeStruct((NW * epw, D), jnp.float32),
            mesh=mesh,
            scratch_types=[
                pltpu.VMEM((epw,), jnp.int32),
                pltpu.VMEM((epw,), jnp.int32),
                pltpu.VMEM((CH, D), jnp.float32),
                pltpu.VMEM((CH, D), jnp.float32),
                pltpu.VMEM((CH, D), jnp.float32),
                pltpu.VMEM((CH, D), jnp.float32),
                pltpu.VMEM((CH, D), jnp.float32),
                pltpu.VMEM((CH, D), jnp.float32),
                pltpu.SemaphoreType.DMA,
                pltpu.SemaphoreType.DMA,
                pltpu.SemaphoreType.DMA,
                pltpu.SemaphoreType.DMA,
            ],
        )(p, q, src, dst)

    BE = 2560

    def edge_mlp_call(x1, ecount):
        return pl.pallas_call(
            _edge_mlp_body,
            grid=(ecount // BE,),
            in_specs=[
                pl.BlockSpec((BE, D), lambda i: (i, 0)),
                _full((D, D)), _full((1, D)), _full((D, D)), _full((1, D)),
            ],
            out_specs=pl.BlockSpec((BE, D), lambda i: (i, 0)),
            out_shape=jax.ShapeDtypeStruct((ecount, D), jnp.float32),
        )(x1, We2.astype(jnp.bfloat16), be2r, We3.astype(jnp.bfloat16), be3r)

    def scatter_call(msg, e0, epw, nch):
        return pl.kernel(
            _make_scatter_body(e0, epw, nch),
            out_type=jax.ShapeDtypeStruct((NC, VP, D), jnp.float32),
            mesh=mesh,
            scratch_types=[
                pltpu.VMEM((CH,), jnp.int32),
                pltpu.VMEM((CH,), jnp.int32),
                pltpu.VMEM((CH, D), jnp.float32),
                pltpu.VMEM((CH, D), jnp.float32),
                pltpu.VMEM((ZR, D), jnp.float32),
                pltpu.VMEM_SHARED((VP, D), jnp.float32),
                pltpu.SemaphoreType.DMA,
                pltpu.SemaphoreType.DMA,
                pltpu.SemaphoreType.DMA,
                pltpu.SemaphoreType.DMA,
            ],
        )(msg, dst)

    x1s = [gather_call(e0, epw, nch)
           for e0, epw, nch in zip(SPLIT_E0, SPLIT_EPW, SPLIT_NCH)]
    msgs = [edge_mlp_call(x1, ecount) for x1, ecount in zip(x1s, SPLIT_E)]
    aggps = [scatter_call(m, e0, epw, nch)
             for m, e0, epw, nch in
             zip(msgs, SPLIT_E0, SPLIT_EPW, SPLIT_NCH)]

    agg_specs = []
    agg_args = []
    for a in aggps:
        agg_specs.append(pl.BlockSpec((1, BV, D), lambda i: (0, i, 0)))
        agg_specs.append(pl.BlockSpec((1, BV, D), lambda i: (1, i, 0)))
        agg_args.extend([a, a])
    out = pl.pallas_call(
        _make_node_mlp_body(len(agg_args)),
        grid=(V // BV,),
        in_specs=[
            pl.BlockSpec((BV, D), lambda i: (i, 0)),
            *agg_specs,
            _full((D, D)), _full((D, D)), _full((1, D)),
            _full((D, D)), _full((1, D)),
        ],
        out_specs=pl.BlockSpec((BV, D), lambda i: (i, 0)),
        out_shape=jax.ShapeDtypeStruct((V, D), jnp.float32),
    )(h, *agg_args, Wn1[:D], Wn1[D:], bn1r, Wn2, bn2r)
    return out


# speedup vs baseline: 1.0250x; 1.0250x over previous
import functools

import jax
import jax.numpy as jnp
from jax import lax
from jax.experimental import pallas as pl
from jax.experimental.pallas import tpu as pltpu
from jax.experimental.pallas import tpu_sc as plsc

V = 10000
E = 320000
D = 128

NC = 2
NS = 16
NW = NC * NS
EPW = E // NW
CH = 80
NCH = EPW // CH
SPLIT_NCH = (15, 40, 40, 30)
SPLIT_EPW = tuple(n * CH for n in SPLIT_NCH)
SPLIT_E = tuple(NW * epw for epw in SPLIT_EPW)
SPLIT_E0 = tuple(sum(SPLIT_E[:i]) for i in range(len(SPLIT_E)))
VP = 10240
VPS = VP // NS
ZR = 128


def _proj_body(h_ref, wa_ref, wb_ref, be1_ref, p_ref, q_ref):
    h = h_ref[...]
    p_ref[...] = jnp.dot(h, wa_ref[...], preferred_element_type=jnp.float32)
    q_ref[...] = (jnp.dot(h, wb_ref[...], preferred_element_type=jnp.float32)
                  + be1_ref[...])


def _edge_mlp_body(x1_ref, w2_ref, b2_ref, w3_ref, b3_ref, msg_ref):
    x1 = x1_ref[...].astype(jnp.bfloat16)
    x2 = jnp.maximum(
        jnp.dot(x1, w2_ref[...], preferred_element_type=jnp.float32)
        + b2_ref[...], 0.0).astype(jnp.bfloat16)
    msg_ref[...] = (jnp.dot(x2, w3_ref[...], preferred_element_type=jnp.float32)
                    + b3_ref[...])


def _make_node_mlp_body(nagg):
    def body(h_ref, *refs):
        aggs = refs[:nagg]
        wna_ref, wnb_ref, bn1_ref, wn2_ref, bn2_ref, out_ref = refs[nagg:]
        agg = aggs[0][0]
        for a in aggs[1:]:
            agg = agg + a[0]
        _node_mlp_core(h_ref, agg, wna_ref, wnb_ref, bn1_ref, wn2_ref,
                       bn2_ref, out_ref)
    return body


def _node_mlp_core(h_ref, agg, wna_ref, wnb_ref, bn1_ref,
                   wn2_ref, bn2_ref, out_ref):
    y = jnp.maximum(
        jnp.dot(h_ref[...], wna_ref[...], preferred_element_type=jnp.float32)
        + jnp.dot(agg, wnb_ref[...], preferred_element_type=jnp.float32)
        + bn1_ref[...], 0.0)
    out_ref[...] = (jnp.dot(y, wn2_ref[...], preferred_element_type=jnp.float32)
                    + bn2_ref[...])


def _full(shape):
    return pl.BlockSpec(shape, lambda i: (0,) * len(shape))


def _make_gather_body(e0, epw, nch):
    def body(p_hbm, q_hbm, src_hbm, dst_hbm, x1_hbm,
             sidx, didx, pbuf0, qbuf0, pbuf1, qbuf1, obuf0, obuf1,
             gsem0, gsem1, wsem0, wsem1):
        c = lax.axis_index("c")
        s = lax.axis_index("s")
        wid = s * NC + c
        gbase = e0 + wid * epw
        lbase = wid * epw
        pbufs, qbufs = (pbuf0, pbuf1), (qbuf0, qbuf1)
        obufs = (obuf0, obuf1)
        gsems, wsems = (gsem0, gsem1), (wsem0, wsem1)

        pltpu.sync_copy(src_hbm.at[pl.ds(gbase, epw)], sidx)
        pltpu.sync_copy(dst_hbm.at[pl.ds(gbase, epw)], didx)

        def fire_gather(ch, b):
            pltpu.async_copy(p_hbm.at[sidx.at[pl.ds(ch * CH, CH)]], pbufs[b],
                             gsems[b])
            pltpu.async_copy(q_hbm.at[didx.at[pl.ds(ch * CH, CH)]], qbufs[b],
                             gsems[b])

        def wait_gather(b):
            pltpu.make_async_copy(p_hbm.at[sidx.at[pl.ds(0, CH)]], pbufs[b],
                                  gsems[b]).wait()
            pltpu.make_async_copy(q_hbm.at[didx.at[pl.ds(0, CH)]], qbufs[b],
                                  gsems[b]).wait()

        def fire_wb(ch, b):
            off = lbase + ch * CH
            pltpu.async_copy(obufs[b], x1_hbm.at[pl.ds(off, CH)], wsems[b])

        def wait_wb(b):
            pltpu.make_async_copy(obufs[b], x1_hbm.at[pl.ds(lbase, CH)],
                                  wsems[b]).wait()

        def compute(b):
            pb, qb, ob = pbufs[b], qbufs[b], obufs[b]

            def row(r, carry):
                for j in range(D // 16):
                    sl = pl.ds(j * 16, 16)
                    ob[r, sl] = jnp.maximum(pb[r, sl] + qb[r, sl], 0.0)
                return carry

            lax.fori_loop(0, CH, row, 0)

        fire_gather(0, 0)

        def super_step(i, carry):
            for b in range(2):
                ch = 2 * i + b
                wait_gather(b)
                fire_gather(ch + 1, 1 - b)

                @pl.when(ch >= 2)
                def _():
                    wait_wb(b)

                compute(b)
                fire_wb(ch, b)
            return carry

        lax.fori_loop(0, (nch - 1) // 2, super_step, 0)
        if nch % 2 == 1:
            wait_gather(0)
            wait_wb(0)
            compute(0)
            fire_wb(nch - 1, 0)
            wait_wb(1)
            wait_wb(0)
        else:
            wait_gather(0)
            fire_gather(nch - 1, 1)
            wait_wb(0)
            compute(0)
            fire_wb(nch - 2, 0)
            wait_gather(1)
            wait_wb(1)
            compute(1)
            fire_wb(nch - 1, 1)
            wait_wb(0)
            wait_wb(1)

    return body


def _make_scatter_body(e0, epw, nch):
    def body(msg_hbm, dst_hbm, out_hbm,
             didx0, didx1, mbuf0, mbuf1, zbuf, agg_sh,
             lsem0, lsem1, ssem0, ssem1):
        c = lax.axis_index("c")
        s = lax.axis_index("s")
        wid = s * NC + c
        gbase = e0 + wid * epw
        lbase = wid * epw
        didxs, mbufs = (didx0, didx1), (mbuf0, mbuf1)
        lsems, ssems = (lsem0, lsem1), (ssem0, ssem1)

        def fire_load(ch, b):
            pltpu.async_copy(dst_hbm.at[pl.ds(gbase + ch * CH, CH)],
                             didxs[b], lsems[b])
            pltpu.async_copy(msg_hbm.at[pl.ds(lbase + ch * CH, CH)],
                             mbufs[b], lsems[b])

        def wait_load(b):
            pltpu.make_async_copy(dst_hbm.at[pl.ds(gbase, CH)], didxs[b],
                                  lsems[b]).wait()
            pltpu.make_async_copy(msg_hbm.at[pl.ds(lbase, CH)], mbufs[b],
                                  lsems[b]).wait()

        def fire_scat(b):
            pltpu.async_copy(mbufs[b], agg_sh.at[didxs[b]], ssems[b],
                             add=True)

        def wait_scat(b):
            pltpu.make_async_copy(mbufs[b], agg_sh.at[didxs[b]],
                                  ssems[b]).wait()

        zero = jnp.zeros((16,), jnp.float32)

        def zrow(i, carry):
            for j in range(D // 16):
                zbuf[i, pl.ds(j * 16, 16)] = zero
            return carry

        fire_load(0, 0)
        lax.fori_loop(0, ZR, zrow, 0)
        for k in range(VPS // ZR):
            pltpu.sync_copy(zbuf, agg_sh.at[pl.ds(s * VPS + k * ZR, ZR)])
        plsc.subcore_barrier()

        def super_step(i, carry):
            for b in range(2):
                ch = 2 * i + b
                wait_load(b)
                fire_scat(b)

                @pl.when(ch >= 1)
                def _():
                    wait_scat(1 - b)

                fire_load(ch + 1, 1 - b)
            return carry

        lax.fori_loop(0, (nch - 1) // 2, super_step, 0)
        if nch % 2 == 1:
            wait_load(0)
            fire_scat(0)
            wait_scat(1)
            wait_scat(0)
        else:
            wait_load(0)
            fire_scat(0)
            wait_scat(1)
            fire_load(nch - 1, 1)
            wait_load(1)
            fire_scat(1)
            wait_scat(0)
            wait_scat(1)
        plsc.subcore_barrier()
        pltpu.sync_copy(agg_sh.at[pl.ds(s * VPS, VPS)],
                        out_hbm.at[c, pl.ds(s * VPS, VPS)])

    return body


@jax.jit
def kernel(h, edge_index, We1, be1, We2, be2, We3, be3, Wn1, bn1, Wn2, bn2):
    src = edge_index[0].astype(jnp.int32)
    dst = edge_index[1].astype(jnp.int32)
    be1r = be1.reshape(1, D)
    be2r = be2.reshape(1, D)
    be3r = be3.reshape(1, D)
    bn1r = bn1.reshape(1, D)
    bn2r = bn2.reshape(1, D)

    BV = 1000
    p, q = pl.pallas_call(
        _proj_body,
        grid=(V // BV,),
        in_specs=[
            pl.BlockSpec((BV, D), lambda i: (i, 0)),
            _full((D, D)), _full((D, D)), _full((1, D)),
        ],
        out_specs=[
            pl.BlockSpec((BV, D), lambda i: (i, 0)),
            pl.BlockSpec((BV, D), lambda i: (i, 0)),
        ],
        out_shape=[
            jax.ShapeDtypeStruct((V, D), jnp.float32),
            jax.ShapeDtypeStruct((V, D), jnp.float32),
        ],
    )(h, We1[:D], We1[D:], be1r)

    mesh = plsc.VectorSubcoreMesh(core_axis_name="c", subcore_axis_name="s")

    def gather_call(e0, epw, nch):
        return pl.kernel(
            _make_gather_body(e0, epw, nch),
            out_type=jax.ShapeDtypeStruct((NW * epw, D), jnp.float32),
            mesh=mesh,
            scratch_types=[
                pltpu.VMEM((epw,), jnp.int32),
                pltpu.VMEM((epw,), jnp.int32),
                pltpu.VMEM((CH, D), jnp.float32),
                pltpu.VMEM((CH, D), jnp.float32),
                pltpu.VMEM((CH, D), jnp.float32),
                pltpu.VMEM((CH, D), jnp.float32),
                pltpu.VMEM((CH, D), jnp.float32),
                pltpu.VMEM((CH, D), jnp.float32),
                pltpu.SemaphoreType.DMA,
                pltpu.SemaphoreType.DMA,
                pltpu.SemaphoreType.DMA,
                pltpu.SemaphoreType.DMA,
            ],
        )(p, q, src, dst)

    BE = 1280

    def edge_mlp_call(x1, ecount):
        return pl.pallas_call(
            _edge_mlp_body,
            grid=(ecount // BE,),
            in_specs=[
                pl.BlockSpec((BE, D), lambda i: (i, 0)),
                _full((D, D)), _full((1, D)), _full((D, D)), _full((1, D)),
            ],
            out_specs=pl.BlockSpec((BE, D), lambda i: (i, 0)),
            out_shape=jax.ShapeDtypeStruct((ecount, D), jnp.float32),
        )(x1, We2.astype(jnp.bfloat16), be2r, We3.astype(jnp.bfloat16), be3r)

    def scatter_call(msg, e0, epw, nch):
        return pl.kernel(
            _make_scatter_body(e0, epw, nch),
            out_type=jax.ShapeDtypeStruct((NC, VP, D), jnp.float32),
            mesh=mesh,
            scratch_types=[
                pltpu.VMEM((CH,), jnp.int32),
                pltpu.VMEM((CH,), jnp.int32),
                pltpu.VMEM((CH, D), jnp.float32),
                pltpu.VMEM((CH, D), jnp.float32),
                pltpu.VMEM((ZR, D), jnp.float32),
                pltpu.VMEM_SHARED((VP, D), jnp.float32),
                pltpu.SemaphoreType.DMA,
                pltpu.SemaphoreType.DMA,
                pltpu.SemaphoreType.DMA,
                pltpu.SemaphoreType.DMA,
            ],
        )(msg, dst)

    x1s = [gather_call(e0, epw, nch)
           for e0, epw, nch in zip(SPLIT_E0, SPLIT_EPW, SPLIT_NCH)]
    msgs = [edge_mlp_call(x1, ecount) for x1, ecount in zip(x1s, SPLIT_E)]
    aggps = [scatter_call(m, e0, epw, nch)
             for m, e0, epw, nch in
             zip(msgs, SPLIT_E0, SPLIT_EPW, SPLIT_NCH)]

    agg_specs = []
    agg_args = []
    for a in aggps:
        agg_specs.append(pl.BlockSpec((1, BV, D), lambda i: (0, i, 0)))
        agg_specs.append(pl.BlockSpec((1, BV, D), lambda i: (1, i, 0)))
        agg_args.extend([a, a])
    out = pl.pallas_call(
        _make_node_mlp_body(len(agg_args)),
        grid=(V // BV,),
        in_specs=[
            pl.BlockSpec((BV, D), lambda i: (i, 0)),
            *agg_specs,
            _full((D, D)), _full((D, D)), _full((1, D)),
            _full((D, D)), _full((1, D)),
        ],
        out_specs=pl.BlockSpec((BV, D), lambda i: (i, 0)),
        out_shape=jax.ShapeDtypeStruct((V, D), jnp.float32),
    )(h, *agg_args, Wn1[:D], Wn1[D:], bn1r, Wn2, bn2r)
    return out


# speedup vs baseline: 1.0271x; 1.0021x over previous
import functools

import jax
import jax.numpy as jnp
from jax import lax
from jax.experimental import pallas as pl
from jax.experimental.pallas import tpu as pltpu
from jax.experimental.pallas import tpu_sc as plsc

V = 10000
E = 320000
D = 128

NC = 2
NS = 16
NW = NC * NS
EPW = E // NW
CH = 80
NCH = EPW // CH
SPLIT_NCH = (15, 45, 45, 20)
SPLIT_EPW = tuple(n * CH for n in SPLIT_NCH)
SPLIT_E = tuple(NW * epw for epw in SPLIT_EPW)
SPLIT_E0 = tuple(sum(SPLIT_E[:i]) for i in range(len(SPLIT_E)))
VP = 10240
VPS = VP // NS
ZR = 128


def _proj_body(h_ref, wa_ref, wb_ref, be1_ref, p_ref, q_ref):
    h = h_ref[...]
    p_ref[...] = jnp.dot(h, wa_ref[...], preferred_element_type=jnp.float32)
    q_ref[...] = (jnp.dot(h, wb_ref[...], preferred_element_type=jnp.float32)
                  + be1_ref[...])


def _edge_mlp_body(x1_ref, w2_ref, b2_ref, w3_ref, b3_ref, msg_ref):
    x1 = x1_ref[...].astype(jnp.bfloat16)
    x2 = jnp.maximum(
        jnp.dot(x1, w2_ref[...], preferred_element_type=jnp.float32)
        + b2_ref[...], 0.0).astype(jnp.bfloat16)
    msg_ref[...] = (jnp.dot(x2, w3_ref[...], preferred_element_type=jnp.float32)
                    + b3_ref[...])


def _make_node_mlp_body(nagg):
    def body(h_ref, *refs):
        aggs = refs[:nagg]
        wna_ref, wnb_ref, bn1_ref, wn2_ref, bn2_ref, out_ref = refs[nagg:]
        agg = aggs[0][0]
        for a in aggs[1:]:
            agg = agg + a[0]
        _node_mlp_core(h_ref, agg, wna_ref, wnb_ref, bn1_ref, wn2_ref,
                       bn2_ref, out_ref)
    return body


def _node_mlp_core(h_ref, agg, wna_ref, wnb_ref, bn1_ref,
                   wn2_ref, bn2_ref, out_ref):
    y = jnp.maximum(
        jnp.dot(h_ref[...], wna_ref[...], preferred_element_type=jnp.float32)
        + jnp.dot(agg, wnb_ref[...], preferred_element_type=jnp.float32)
        + bn1_ref[...], 0.0)
    out_ref[...] = (jnp.dot(y, wn2_ref[...], preferred_element_type=jnp.float32)
                    + bn2_ref[...])


def _full(shape):
    return pl.BlockSpec(shape, lambda i: (0,) * len(shape))


def _make_gather_body(e0, epw, nch):
    def body(p_hbm, q_hbm, src_hbm, dst_hbm, x1_hbm,
             sidx, didx, pbuf0, qbuf0, pbuf1, qbuf1, obuf0, obuf1,
             gsem0, gsem1, wsem0, wsem1):
        c = lax.axis_index("c")
        s = lax.axis_index("s")
        wid = s * NC + c
        gbase = e0 + wid * epw
        lbase = wid * epw
        pbufs, qbufs = (pbuf0, pbuf1), (qbuf0, qbuf1)
        obufs = (obuf0, obuf1)
        gsems, wsems = (gsem0, gsem1), (wsem0, wsem1)

        pltpu.sync_copy(src_hbm.at[pl.ds(gbase, epw)], sidx)
        pltpu.sync_copy(dst_hbm.at[pl.ds(gbase, epw)], didx)

        def fire_gather(ch, b):
            pltpu.async_copy(p_hbm.at[sidx.at[pl.ds(ch * CH, CH)]], pbufs[b],
                             gsems[b])
            pltpu.async_copy(q_hbm.at[didx.at[pl.ds(ch * CH, CH)]], qbufs[b],
                             gsems[b])

        def wait_gather(b):
            pltpu.make_async_copy(p_hbm.at[sidx.at[pl.ds(0, CH)]], pbufs[b],
                                  gsems[b]).wait()
            pltpu.make_async_copy(q_hbm.at[didx.at[pl.ds(0, CH)]], qbufs[b],
                                  gsems[b]).wait()

        def fire_wb(ch, b):
            off = lbase + ch * CH
            pltpu.async_copy(obufs[b], x1_hbm.at[pl.ds(off, CH)], wsems[b])

        def wait_wb(b):
            pltpu.make_async_copy(obufs[b], x1_hbm.at[pl.ds(lbase, CH)],
                                  wsems[b]).wait()

        def compute(b):
            pb, qb, ob = pbufs[b], qbufs[b], obufs[b]

            def row(r, carry):
                for j in range(D // 16):
                    sl = pl.ds(j * 16, 16)
                    ob[r, sl] = jnp.maximum(pb[r, sl] + qb[r, sl], 0.0)
                return carry

            lax.fori_loop(0, CH, row, 0)

        fire_gather(0, 0)

        def super_step(i, carry):
            for b in range(2):
                ch = 2 * i + b
                wait_gather(b)
                fire_gather(ch + 1, 1 - b)

                @pl.when(ch >= 2)
                def _():
                    wait_wb(b)

                compute(b)
                fire_wb(ch, b)
            return carry

        lax.fori_loop(0, (nch - 1) // 2, super_step, 0)
        if nch % 2 == 1:
            wait_gather(0)
            wait_wb(0)
            compute(0)
            fire_wb(nch - 1, 0)
            wait_wb(1)
            wait_wb(0)
        else:
            wait_gather(0)
            fire_gather(nch - 1, 1)
            wait_wb(0)
            compute(0)
            fire_wb(nch - 2, 0)
            wait_gather(1)
            wait_wb(1)
            compute(1)
            fire_wb(nch - 1, 1)
            wait_wb(0)
            wait_wb(1)

    return body


def _make_scatter_body(e0, epw, nch):
    def body(msg_hbm, dst_hbm, out_hbm,
             didx0, didx1, mbuf0, mbuf1, zbuf, agg_sh,
             lsem0, lsem1, ssem0, ssem1):
        c = lax.axis_index("c")
        s = lax.axis_index("s")
        wid = s * NC + c
        gbase = e0 + wid * epw
        lbase = wid * epw
        didxs, mbufs = (didx0, didx1), (mbuf0, mbuf1)
        lsems, ssems = (lsem0, lsem1), (ssem0, ssem1)

        def fire_load(ch, b):
            pltpu.async_copy(dst_hbm.at[pl.ds(gbase + ch * CH, CH)],
                             didxs[b], lsems[b])
            pltpu.async_copy(msg_hbm.at[pl.ds(lbase + ch * CH, CH)],
                             mbufs[b], lsems[b])

        def wait_load(b):
            pltpu.make_async_copy(dst_hbm.at[pl.ds(gbase, CH)], didxs[b],
                                  lsems[b]).wait()
            pltpu.make_async_copy(msg_hbm.at[pl.ds(lbase, CH)], mbufs[b],
                                  lsems[b]).wait()

        def fire_scat(b):
            pltpu.async_copy(mbufs[b], agg_sh.at[didxs[b]], ssems[b],
                             add=True)

        def wait_scat(b):
            pltpu.make_async_copy(mbufs[b], agg_sh.at[didxs[b]],
                                  ssems[b]).wait()

        zero = jnp.zeros((16,), jnp.float32)

        def zrow(i, carry):
            for j in range(D // 16):
                zbuf[i, pl.ds(j * 16, 16)] = zero
            return carry

        fire_load(0, 0)
        lax.fori_loop(0, ZR, zrow, 0)
        for k in range(VPS // ZR):
            pltpu.sync_copy(zbuf, agg_sh.at[pl.ds(s * VPS + k * ZR, ZR)])
        plsc.subcore_barrier()

        def super_step(i, carry):
            for b in range(2):
                ch = 2 * i + b
                wait_load(b)
                fire_scat(b)

                @pl.when(ch >= 1)
                def _():
                    wait_scat(1 - b)

                fire_load(ch + 1, 1 - b)
            return carry

        lax.fori_loop(0, (nch - 1) // 2, super_step, 0)
        if nch % 2 == 1:
            wait_load(0)
            fire_scat(0)
            wait_scat(1)
            wait_scat(0)
        else:
            wait_load(0)
            fire_scat(0)
            wait_scat(1)
            fire_load(nch - 1, 1)
            wait_load(1)
            fire_scat(1)
            wait_scat(0)
            wait_scat(1)
        plsc.subcore_barrier()
        pltpu.sync_copy(agg_sh.at[pl.ds(s * VPS, VPS)],
                        out_hbm.at[c, pl.ds(s * VPS, VPS)])

    return body


@jax.jit
def kernel(h, edge_index, We1, be1, We2, be2, We3, be3, Wn1, bn1, Wn2, bn2):
    src = edge_index[0].astype(jnp.int32)
    dst = edge_index[1].astype(jnp.int32)
    be1r = be1.reshape(1, D)
    be2r = be2.reshape(1, D)
    be3r = be3.reshape(1, D)
    bn1r = bn1.reshape(1, D)
    bn2r = bn2.reshape(1, D)

    BV = 1000
    p, q = pl.pallas_call(
        _proj_body,
        grid=(V // BV,),
        in_specs=[
            pl.BlockSpec((BV, D), lambda i: (i, 0)),
            _full((D, D)), _full((D, D)), _full((1, D)),
        ],
        out_specs=[
            pl.BlockSpec((BV, D), lambda i: (i, 0)),
            pl.BlockSpec((BV, D), lambda i: (i, 0)),
        ],
        out_shape=[
            jax.ShapeDtypeStruct((V, D), jnp.float32),
            jax.ShapeDtypeStruct((V, D), jnp.float32),
        ],
    )(h, We1[:D], We1[D:], be1r)

    mesh = plsc.VectorSubcoreMesh(core_axis_name="c", subcore_axis_name="s")

    def gather_call(e0, epw, nch):
        return pl.kernel(
            _make_gather_body(e0, epw, nch),
            out_type=jax.ShapeDtypeStruct((NW * epw, D), jnp.float32),
            mesh=mesh,
            scratch_types=[
                pltpu.VMEM((epw,), jnp.int32),
                pltpu.VMEM((epw,), jnp.int32),
                pltpu.VMEM((CH, D), jnp.float32),
                pltpu.VMEM((CH, D), jnp.float32),
                pltpu.VMEM((CH, D), jnp.float32),
                pltpu.VMEM((CH, D), jnp.float32),
                pltpu.VMEM((CH, D), jnp.float32),
                pltpu.VMEM((CH, D), jnp.float32),
                pltpu.SemaphoreType.DMA,
                pltpu.SemaphoreType.DMA,
                pltpu.SemaphoreType.DMA,
                pltpu.SemaphoreType.DMA,
            ],
        )(p, q, src, dst)

    BE = 1280

    def edge_mlp_call(x1, ecount):
        return pl.pallas_call(
            _edge_mlp_body,
            grid=(ecount // BE,),
            in_specs=[
                pl.BlockSpec((BE, D), lambda i: (i, 0)),
                _full((D, D)), _full((1, D)), _full((D, D)), _full((1, D)),
            ],
            out_specs=pl.BlockSpec((BE, D), lambda i: (i, 0)),
            out_shape=jax.ShapeDtypeStruct((ecount, D), jnp.float32),
        )(x1, We2.astype(jnp.bfloat16), be2r, We3.astype(jnp.bfloat16), be3r)

    def scatter_call(msg, e0, epw, nch):
        return pl.kernel(
            _make_scatter_body(e0, epw, nch),
            out_type=jax.ShapeDtypeStruct((NC, VP, D), jnp.float32),
            mesh=mesh,
            scratch_types=[
                pltpu.VMEM((CH,), jnp.int32),
                pltpu.VMEM((CH,), jnp.int32),
                pltpu.VMEM((CH, D), jnp.float32),
                pltpu.VMEM((CH, D), jnp.float32),
                pltpu.VMEM((ZR, D), jnp.float32),
                pltpu.VMEM_SHARED((VP, D), jnp.float32),
                pltpu.SemaphoreType.DMA,
                pltpu.SemaphoreType.DMA,
                pltpu.SemaphoreType.DMA,
                pltpu.SemaphoreType.DMA,
            ],
        )(msg, dst)

    x1s = [gather_call(e0, epw, nch)
           for e0, epw, nch in zip(SPLIT_E0, SPLIT_EPW, SPLIT_NCH)]
    msgs = [edge_mlp_call(x1, ecount) for x1, ecount in zip(x1s, SPLIT_E)]
    aggps = [scatter_call(m, e0, epw, nch)
             for m, e0, epw, nch in
             zip(msgs, SPLIT_E0, SPLIT_EPW, SPLIT_NCH)]

    agg_specs = []
    agg_args = []
    for a in aggps:
        agg_specs.append(pl.BlockSpec((1, BV, D), lambda i: (0, i, 0)))
        agg_specs.append(pl.BlockSpec((1, BV, D), lambda i: (1, i, 0)))
        agg_args.extend([a, a])
    out = pl.pallas_call(
        _make_node_mlp_body(len(agg_args)),
        grid=(V // BV,),
        in_specs=[
            pl.BlockSpec((BV, D), lambda i: (i, 0)),
            *agg_specs,
            _full((D, D)), _full((D, D)), _full((1, D)),
            _full((D, D)), _full((1, D)),
        ],
        out_specs=pl.BlockSpec((BV, D), lambda i: (i, 0)),
        out_shape=jax.ShapeDtypeStruct((V, D), jnp.float32),
    )(h, *agg_args, Wn1[:D], Wn1[D:], bn1r, Wn2, bn2r)
    return out
